# baseline (speedup 1.0000x reference)
import functools
import jax
import jax.numpy as jnp
from jax.experimental import pallas as pl
from jax.experimental.pallas import tpu as pltpu


def _pool_body(seg_ref, m1_ref, m2_ref, w1_ref, w2_ref,
               num1_ref, den1_ref, num2_ref, den2_ref, *, bp):
    i = pl.program_id(0)

    @pl.when(i == 0)
    def _init():
        num1_ref[...] = jnp.zeros_like(num1_ref)
        den1_ref[...] = jnp.zeros_like(den1_ref)
        num2_ref[...] = jnp.zeros_like(num2_ref)
        den2_ref[...] = jnp.zeros_like(den2_ref)

    seg = seg_ref[0, 0, :]
    bn = seg.shape[0]
    oh = (seg[:, None] ==
          jax.lax.broadcasted_iota(jnp.int32, (bn, bp), 1)).astype(jnp.float32)
    oht = oh.T

    m1 = m1_ref[...]
    g1 = jnp.dot(m1, w1_ref[...], preferred_element_type=jnp.float32)
    e1 = jnp.exp(g1)
    num1_ref[...] += jnp.dot(oht, m1 * e1, preferred_element_type=jnp.float32)
    den1_ref[...] += jnp.dot(oht, e1, preferred_element_type=jnp.float32)

    m2 = m2_ref[...]
    g2 = jnp.dot(m2, w2_ref[...], preferred_element_type=jnp.float32)
    e2 = jnp.exp(g2)
    num2_ref[...] += jnp.dot(oht, m2 * e2, preferred_element_type=jnp.float32)
    den2_ref[...] += jnp.dot(oht, e2, preferred_element_type=jnp.float32)


def _att_body(q_ref, num1_ref, den1_ref, num2_ref, den2_ref, att_ref):
    q = q_ref[...]
    den1 = den1_ref[...]
    den2 = den2_ref[...]
    k1 = num1_ref[...] / jnp.where(den1 > 0, den1, 1.0)
    k2 = num2_ref[...] / jnp.where(den2 > 0, den2, 1.0)
    s1 = jnp.sum(q * k1, axis=1, keepdims=True)
    s2 = jnp.sum(q * k2, axis=1, keepdims=True)
    m = jnp.maximum(s1, s2)
    e1 = jnp.exp(s1 - m)
    e2 = jnp.exp(s2 - m)
    tot = e1 + e2
    att_ref[...] = jnp.concatenate([e1 / tot, e2 / tot], axis=1)


def _mix_body(seg_ref, m1_ref, m2_ref, att_ref, out_ref, *, bp):
    seg = seg_ref[0, 0, :]
    bn = seg.shape[0]
    oh = (seg[:, None] ==
          jax.lax.broadcasted_iota(jnp.int32, (bn, bp), 1)).astype(jnp.float32)
    attn = jnp.dot(oh, att_ref[...], preferred_element_type=jnp.float32)
    out_ref[...] = (m1_ref[...] * attn[:, 0:1] + m2_ref[...] * attn[:, 1:2])


def kernel(Q, M1, M2, segment_ids, W1, b1, W2, b2):
    del b1, b2
    n, d = M1.shape
    b = Q.shape[0]
    bp = 64
    bn = 10000
    nb = n // bn
    assert nb * bn == n
    bm = 5000
    nm = n // bm
    assert nm * bm == n

    seg3 = segment_ids.reshape(nb, 1, bn)
    seg3m = segment_ids.reshape(nm, 1, bm)
    f32 = jnp.float32

    num1, den1, num2, den2 = pl.pallas_call(
        functools.partial(_pool_body, bp=bp),
        grid=(nb,),
        in_specs=[
            pl.BlockSpec((1, 1, bn), lambda i: (i, 0, 0)),
            pl.BlockSpec((bn, d), lambda i: (i, 0)),
            pl.BlockSpec((bn, d), lambda i: (i, 0)),
            pl.BlockSpec((d, 1), lambda i: (0, 0)),
            pl.BlockSpec((d, 1), lambda i: (0, 0)),
        ],
        out_specs=[
            pl.BlockSpec((bp, d), lambda i: (0, 0)),
            pl.BlockSpec((bp, 1), lambda i: (0, 0)),
            pl.BlockSpec((bp, d), lambda i: (0, 0)),
            pl.BlockSpec((bp, 1), lambda i: (0, 0)),
        ],
        out_shape=[
            jax.ShapeDtypeStruct((bp, d), f32),
            jax.ShapeDtypeStruct((bp, 1), f32),
            jax.ShapeDtypeStruct((bp, d), f32),
            jax.ShapeDtypeStruct((bp, 1), f32),
        ],
    )(seg3, M1, M2, W1, W2)

    qp = jnp.zeros((bp, d), f32).at[:b].set(Q)
    att = pl.pallas_call(
        _att_body,
        out_shape=jax.ShapeDtypeStruct((bp, 2), f32),
    )(qp, num1, den1, num2, den2)

    out = pl.pallas_call(
        functools.partial(_mix_body, bp=bp),
        grid=(nm,),
        in_specs=[
            pl.BlockSpec((1, 1, bm), lambda i: (i, 0, 0)),
            pl.BlockSpec((bm, d), lambda i: (i, 0)),
            pl.BlockSpec((bm, d), lambda i: (i, 0)),
            pl.BlockSpec((bp, 2), lambda i: (0, 0)),
        ],
        out_specs=pl.BlockSpec((bm, d), lambda i: (i, 0)),
        out_shape=jax.ShapeDtypeStruct((n, d), f32),
    )(seg3m, M1, M2, att)
    return out


# speedup vs baseline: 1.0215x; 1.0215x over previous
import functools
import jax
import jax.numpy as jnp
from jax.experimental import pallas as pl
from jax.experimental.pallas import tpu as pltpu


def _pool_body(seg_ref, m1_ref, m2_ref, w1_ref, w2_ref,
               num1_ref, den1_ref, num2_ref, den2_ref, *, bp):
    i = pl.program_id(0)

    @pl.when(i == 0)
    def _init():
        num1_ref[...] = jnp.zeros_like(num1_ref)
        den1_ref[...] = jnp.zeros_like(den1_ref)
        num2_ref[...] = jnp.zeros_like(num2_ref)
        den2_ref[...] = jnp.zeros_like(den2_ref)

    seg = seg_ref[0, 0, :]
    bn = seg.shape[0]
    oh = (seg[:, None] ==
          jax.lax.broadcasted_iota(jnp.int32, (bn, bp), 1)).astype(jnp.float32)
    oht = oh.T

    m1 = m1_ref[...]
    g1 = jnp.dot(m1, w1_ref[...], preferred_element_type=jnp.float32)
    e1 = jnp.exp(g1)
    num1_ref[...] += jnp.dot(oht, m1 * e1, preferred_element_type=jnp.float32)
    den1_ref[...] += jnp.dot(oht, e1, preferred_element_type=jnp.float32)

    m2 = m2_ref[...]
    g2 = jnp.dot(m2, w2_ref[...], preferred_element_type=jnp.float32)
    e2 = jnp.exp(g2)
    num2_ref[...] += jnp.dot(oht, m2 * e2, preferred_element_type=jnp.float32)
    den2_ref[...] += jnp.dot(oht, e2, preferred_element_type=jnp.float32)


def _att_body(q_ref, num1_ref, den1_ref, num2_ref, den2_ref, att_ref):
    q = q_ref[...]
    den1 = den1_ref[...]
    den2 = den2_ref[...]
    k1 = num1_ref[...] / jnp.where(den1 > 0, den1, 1.0)
    k2 = num2_ref[...] / jnp.where(den2 > 0, den2, 1.0)
    s1 = jnp.sum(q * k1, axis=1, keepdims=True)
    s2 = jnp.sum(q * k2, axis=1, keepdims=True)
    m = jnp.maximum(s1, s2)
    e1 = jnp.exp(s1 - m)
    e2 = jnp.exp(s2 - m)
    tot = e1 + e2
    att_ref[...] = jnp.concatenate([e1 / tot, e2 / tot], axis=1)


def _mix_body(seg_ref, m1_ref, m2_ref, att_ref, out_ref, *, bp):
    seg = seg_ref[0, 0, :]
    bn = seg.shape[0]
    oh = (seg[:, None] ==
          jax.lax.broadcasted_iota(jnp.int32, (bn, bp), 1)).astype(jnp.float32)
    attn = jnp.dot(oh, att_ref[...], preferred_element_type=jnp.float32)
    out_ref[...] = (m1_ref[...] * attn[:, 0:1] + m2_ref[...] * attn[:, 1:2])


def kernel(Q, M1, M2, segment_ids, W1, b1, W2, b2):
    del b1, b2
    n, d = M1.shape
    b = Q.shape[0]
    bp = 64
    bn = 5000
    nb = n // bn
    assert nb * bn == n
    bm = 5000
    nm = n // bm
    assert nm * bm == n

    seg3 = segment_ids.reshape(nb, 1, bn)
    seg3m = segment_ids.reshape(nm, 1, bm)
    f32 = jnp.float32

    num1, den1, num2, den2 = pl.pallas_call(
        functools.partial(_pool_body, bp=bp),
        grid=(nb,),
        in_specs=[
            pl.BlockSpec((1, 1, bn), lambda i: (i, 0, 0)),
            pl.BlockSpec((bn, d), lambda i: (i, 0)),
            pl.BlockSpec((bn, d), lambda i: (i, 0)),
            pl.BlockSpec((d, 1), lambda i: (0, 0)),
            pl.BlockSpec((d, 1), lambda i: (0, 0)),
        ],
        out_specs=[
            pl.BlockSpec((bp, d), lambda i: (0, 0)),
            pl.BlockSpec((bp, 1), lambda i: (0, 0)),
            pl.BlockSpec((bp, d), lambda i: (0, 0)),
            pl.BlockSpec((bp, 1), lambda i: (0, 0)),
        ],
        out_shape=[
            jax.ShapeDtypeStruct((bp, d), f32),
            jax.ShapeDtypeStruct((bp, 1), f32),
            jax.ShapeDtypeStruct((bp, d), f32),
            jax.ShapeDtypeStruct((bp, 1), f32),
        ],
    )(seg3, M1, M2, W1, W2)

    qp = jnp.zeros((bp, d), f32).at[:b].set(Q)
    att = pl.pallas_call(
        _att_body,
        out_shape=jax.ShapeDtypeStruct((bp, 2), f32),
    )(qp, num1, den1, num2, den2)

    out = pl.pallas_call(
        functools.partial(_mix_body, bp=bp),
        grid=(nm,),
        in_specs=[
            pl.BlockSpec((1, 1, bm), lambda i: (i, 0, 0)),
            pl.BlockSpec((bm, d), lambda i: (i, 0)),
            pl.BlockSpec((bm, d), lambda i: (i, 0)),
            pl.BlockSpec((bp, 2), lambda i: (0, 0)),
        ],
        out_specs=pl.BlockSpec((bm, d), lambda i: (i, 0)),
        out_shape=jax.ShapeDtypeStruct((n, d), f32),
    )(seg3m, M1, M2, att)
    return out
